# baseline (speedup 1.0000x reference)
import functools

import jax
import jax.numpy as jnp
from jax import lax
from jax.experimental import pallas as pl
from jax.experimental.pallas import tpu as pltpu
from jax.experimental.pallas import tpu_sc as plsc

NC = 2
NS = 16
NW = NC * NS
L = 16
SCAN = 8192
BCAP = 16384
PADSLOT = BCAP

_mesh = plsc.VectorSubcoreMesh(core_axis_name="c", subcore_axis_name="s")


def _make_sc_agg(E, n_src_pad, n_tgt, D):
  TR = n_tgt // NW
  NVT = TR // L
  TSHIFT = 14 + (TR.bit_length() - 1)
  NCHUNK = E // SCAN
  PAD = n_src_pad - 8

  def body(xp_hbm, pk_hbm, agg_hbm, deg_hbm,
           ebuf0, ebuf1, mine, srt, acc, gb0, gb1, hist, cnt, base, degt,
           idx_a, idx_b, degf, esem0, esem1, sem_a, sem_b):
    c = lax.axis_index("c")
    s = lax.axis_index("s")
    tid = c * NS + s
    iota = lax.iota(jnp.int32, L)
    zi = jnp.zeros((L,), jnp.int32)

    def zrow(i, carry):
      for kb in range(D // L):
        acc[i, pl.ds(kb * L, L)] = jnp.zeros((L,), jnp.float32)
      return carry
    lax.fori_loop(0, TR, zrow, 0)
    for kb in range(NVT):
      degt[pl.ds(kb * L, L)] = zi
      degf[pl.ds(kb * L, L)] = jnp.zeros((L,), jnp.float32)
    srt[pl.ds(PADSLOT, L)] = zi + PAD

    def process_batch(bcnt):
      for kb in range(NVT):
        hist[pl.ds(kb * L, L)] = zi
        cnt[pl.ds(kb * L, L)] = zi

      def occ_terms(k):
        lanes = k * L + iota
        m = lanes < bcnt
        v = mine[pl.ds(k * L, L)]
        dlr = jnp.bitwise_and(lax.shift_right_logical(v, 14), TR - 1)
        sk, sp, om = plsc.sort_key_val(dlr, iota, mask=m)
        spc = jnp.where(om, sp, 0)
        sv = v.at[spc].get(mode="promise_in_bounds")
        om32 = jnp.where(om, 1, 0)
        skp = sk.at[jnp.maximum(iota - 1, 0)].get(mode="promise_in_bounds")
        bstart = (sk != skp) | (iota == 0)
        runstart = plsc.cummax(jnp.where(bstart, iota, 0))
        duprank = iota - runstart
        nxt = jnp.minimum(iota + 1, L - 1)
        skn = sk.at[nxt].get(mode="promise_in_bounds")
        omn = om32.at[nxt].get(mode="promise_in_bounds")
        lastm = om & ((skn != sk) | (omn == 0) | (iota == L - 1))
        total = duprank + 1
        return sv, sk, om, duprank, lastm, total

      nv = (bcnt + L - 1) // L

      def p1(k, carry):
        _, dl, _, _, lastm, total = occ_terms(k)
        plsc.addupdate_scatter(hist, [dl], total, mask=lastm)
        return carry
      lax.fori_loop(0, nv, p1, 0)

      running = jnp.int32(0)
      mv = zi
      for kb in range(NVT):
        hv = hist[pl.ds(kb * L, L)]
        inc = plsc.cumsum(hv)
        base[pl.ds(kb * L, L)] = inc - hv + running
        running = running + inc[15]
        mv = jnp.maximum(mv, hv)
      rmax = lax.reduce_max(mv.astype(jnp.float32), axes=(0,)).astype(jnp.int32)

      def p2(k, carry):
        sv, dl, om, duprank, lastm, total = occ_terms(k)
        g = plsc.load_gather(cnt, [dl], mask=om)
        b = plsc.load_gather(base, [dl], mask=om)
        pos = jnp.minimum(b + g + duprank, BCAP - 1)
        src = jnp.bitwise_and(sv, 16383)
        plsc.store_scatter(srt, [pos], src, mask=om)
        plsc.addupdate_scatter(cnt, [dl], total, mask=lastm)
        return carry
      lax.fori_loop(0, nv, p2, 0)

      def build(r, idxb):
        rv = zi + r
        for kb in range(NVT):
          bv = base[pl.ds(kb * L, L)]
          dv = hist[pl.ds(kb * L, L)]
          slot = jnp.where(rv < dv, bv + rv, PADSLOT)
          idxb[pl.ds(kb * L, L)] = plsc.load_gather(srt, [slot])

      def addinto(gb):
        def arow(i, carry):
          for kb in range(D // L):
            acc[i, pl.ds(kb * L, L)] = (acc[i, pl.ds(kb * L, L)] +
                                        gb[i, pl.ds(kb * L, L)])
          return carry
        lax.fori_loop(0, TR, arow, 0)

      build(jnp.int32(0), idx_a)
      pltpu.async_copy(xp_hbm.at[idx_a], gb0, sem_a)

      def rnd2(i, carry):
        r = 2 * i
        pltpu.make_async_copy(xp_hbm.at[idx_a], gb0, sem_a).wait()
        build(r + 1, idx_b)
        pltpu.async_copy(xp_hbm.at[idx_b], gb1, sem_b)
        addinto(gb0)
        pltpu.make_async_copy(xp_hbm.at[idx_b], gb1, sem_b).wait()
        build(r + 2, idx_a)
        pltpu.async_copy(xp_hbm.at[idx_a], gb0, sem_a)
        addinto(gb1)
        return carry
      lax.fori_loop(0, (jnp.maximum(rmax, 1) + 1) // 2, rnd2, 0)
      pltpu.make_async_copy(xp_hbm.at[idx_a], gb0, sem_a).wait()

      for kb in range(NVT):
        degt[pl.ds(kb * L, L)] = degt[pl.ds(kb * L, L)] + hist[pl.ds(kb * L, L)]
      return jnp.int32(0)

    def scan_chunk(ebuf, mcnt):
      def sv(k, mc):
        v = ebuf[pl.ds(k * L, L)]
        own = lax.shift_right_logical(v, TSHIFT) == tid
        plsc.store_compressed(mine.at[pl.ds(mc, L)], v, mask=own)
        n = plsc.all_reduce_population_count(own)
        return mc + n[0]
      return lax.fori_loop(0, SCAN // L, sv, mcnt)

    pltpu.async_copy(pk_hbm.at[pl.ds(0, SCAN)], ebuf0, esem0)
    mcnt = jnp.int32(0)
    for ch in range(NCHUNK):
      eb = ebuf0 if ch % 2 == 0 else ebuf1
      es = esem0 if ch % 2 == 0 else esem1
      pltpu.make_async_copy(pk_hbm.at[pl.ds(0, SCAN)], eb, es).wait()
      if ch + 1 < NCHUNK:
        nb = ebuf1 if ch % 2 == 0 else ebuf0
        ns = esem1 if ch % 2 == 0 else esem0
        pltpu.async_copy(pk_hbm.at[pl.ds((ch + 1) * SCAN, SCAN)], nb, ns)
      mcnt = scan_chunk(eb, mcnt)
      mcnt = lax.cond(mcnt >= SCAN, process_batch, lambda b: b, mcnt)
    mcnt = process_batch(mcnt)

    for kb in range(NVT):
      degf[pl.ds(kb * L, L)] = degt[pl.ds(kb * L, L)].astype(jnp.float32)

    pltpu.sync_copy(acc, agg_hbm.at[pl.ds(tid * TR, TR)])
    pltpu.sync_copy(degf, deg_hbm.at[pl.ds(tid * TR, TR)])

  return pl.kernel(
      body,
      out_type=(jax.ShapeDtypeStruct((n_tgt, D), jnp.float32),
                jax.ShapeDtypeStruct((n_tgt,), jnp.float32)),
      mesh=_mesh,
      scratch_types=[
          pltpu.VMEM((SCAN,), jnp.int32),
          pltpu.VMEM((SCAN,), jnp.int32),
          pltpu.VMEM((BCAP + L,), jnp.int32),
          pltpu.VMEM((BCAP + L,), jnp.int32),
          pltpu.VMEM((TR, D), jnp.float32),
          pltpu.VMEM((TR, D), jnp.float32),
          pltpu.VMEM((TR, D), jnp.float32),
          pltpu.VMEM((TR,), jnp.int32),
          pltpu.VMEM((TR,), jnp.int32),
          pltpu.VMEM((TR,), jnp.int32),
          pltpu.VMEM((TR,), jnp.int32),
          pltpu.VMEM((TR,), jnp.int32),
          pltpu.VMEM((TR,), jnp.int32),
          pltpu.VMEM((TR,), jnp.float32),
          pltpu.SemaphoreType.DMA,
          pltpu.SemaphoreType.DMA,
          pltpu.SemaphoreType.DMA,
          pltpu.SemaphoreType.DMA,
      ],
      compiler_params=pltpu.CompilerParams(needs_layout_passes=False),
  )


_sc_agg0 = _make_sc_agg(131072, 10008, 4096, 128)
_sc_agg1 = _make_sc_agg(32768, 4104, 1024, 256)


def _tc_body(a_ref, d_ref, w_ref, b_ref, o_ref, *, last):
  deg = d_ref[...]
  m = a_ref[...] / jnp.maximum(deg, 1.0)
  h = jnp.dot(m, w_ref[...], preferred_element_type=jnp.float32) + b_ref[...]
  if last:
    mx = jnp.max(h, axis=-1, keepdims=True)
    lse = jnp.log(jnp.sum(jnp.exp(h - mx), axis=-1, keepdims=True)) + mx
    o_ref[...] = h - lse
  else:
    o_ref[...] = jnp.maximum(h, 0.0)


def _tc_dense(acc, deg, W, b, last):
  return pl.pallas_call(
      functools.partial(_tc_body, last=last),
      out_shape=jax.ShapeDtypeStruct((acc.shape[0], W.shape[1]), jnp.float32),
  )(acc, deg.reshape(-1, 1), W, b.reshape(1, -1))


def kernel(x, adj0, adj1, W0, b0, W1, b1):
  xp = jnp.concatenate([x, jnp.zeros((8, x.shape[1]), x.dtype)], axis=0)
  pk0 = adj0[1] * 16384 + adj0[0]
  pk1 = adj1[1] * 16384 + adj1[0]
  agg0, deg0 = _sc_agg0(xp, pk0)
  h1 = _tc_dense(agg0, deg0, W0, b0, last=False)
  h1p = jnp.concatenate([h1, jnp.zeros((8, h1.shape[1]), h1.dtype)], axis=0)
  agg1, deg1 = _sc_agg1(h1p, pk1)
  return _tc_dense(agg1, deg1, W1, b1, last=True)
